# baseline (device time: 11929 ns/iter reference)
import jax
import jax.numpy as jnp
from jax import lax
from jax.experimental import pallas as pl
from jax.experimental.pallas import tpu as pltpu

N_DEV = 4


def kernel(x, router_W, route_idx, expert_W, shared_W):
    n_tok, d_model = x.shape
    n_local_exp, _, d_ff = expert_W.shape
    n_exp = router_W.shape[1]
    d_cat = n_local_exp * d_model

    ew16 = expert_W.reshape(d_cat, d_ff).astype(jnp.bfloat16)
    sw16 = shared_W.astype(jnp.bfloat16)

    def body(x_ref, rw_ref, idx_ref, ew16_ref, sw16_ref, out_ref,
             comm_ref, send_sems, recv_sems):
        my_pos = lax.axis_index("i")

        barrier_sem = pltpu.get_barrier_semaphore()
        for k in range(1, N_DEV):
            pl.semaphore_signal(
                barrier_sem, inc=1,
                device_id=(lax.rem(my_pos + k, N_DEV),),
                device_id_type=pl.DeviceIdType.MESH,
            )
        pl.semaphore_wait(barrier_sem, N_DEV - 1)

        sends = []
        for k in range(1, N_DEV):
            s = pltpu.make_async_remote_copy(
                src_ref=ew16_ref,
                dst_ref=comm_ref.at[N_DEV - k],
                send_sem=send_sems.at[k - 1],
                recv_sem=recv_sems.at[N_DEV - k],
                device_id=(lax.rem(my_pos + k, N_DEV),),
                device_id_type=pl.DeviceIdType.MESH,
            )
            s.start()
            sends.append(s)

        def recv_for(slot):
            return pltpu.make_async_remote_copy(
                src_ref=ew16_ref,
                dst_ref=comm_ref.at[slot],
                send_sem=send_sems.at[N_DEV - 1],
                recv_sem=recv_sems.at[slot],
                device_id=(my_pos,),
                device_id_type=pl.DeviceIdType.MESH,
            )

        xv = x_ref[...]
        eid = idx_ref[...]
        scores = jnp.dot(xv, rw_ref[...], preferred_element_type=jnp.float32)
        m = jnp.max(scores, axis=-1, keepdims=True)
        p = jnp.exp(scores - m)
        denom = jnp.sum(p, axis=-1, keepdims=True)
        onehot = lax.broadcasted_iota(jnp.int32, (n_tok, n_exp), 1) == eid
        gate = jnp.sum(jnp.where(onehot, p, 0.0), axis=-1, keepdims=True) / denom

        def scaled_x_for(origin):
            parts = []
            for j in range(n_local_exp):
                e = origin * n_local_exp + j
                w = jnp.where(eid == e, gate, 0.0)
                parts.append((xv * w).astype(jnp.bfloat16))
            return jnp.concatenate(parts, axis=1)

        xm = [scaled_x_for(lax.rem(my_pos + s, N_DEV)) for s in range(N_DEV)]

        x16 = xv.astype(jnp.bfloat16)
        acc = jnp.dot(x16, sw16_ref[...], preferred_element_type=jnp.float32)
        acc = acc + jnp.dot(xm[0], ew16_ref[...],
                            preferred_element_type=jnp.float32)

        for slot in (1, 3, 2):
            recv_for(slot).wait_recv()
            acc = acc + jnp.dot(xm[slot], comm_ref[slot],
                                preferred_element_type=jnp.float32)

        for s in sends:
            s.wait_send()

        out_ref[...] = acc

    return pl.pallas_call(
        body,
        out_shape=jax.ShapeDtypeStruct((n_tok, d_ff), jnp.float32),
        in_specs=[pl.BlockSpec(memory_space=pltpu.VMEM)] * 5,
        out_specs=pl.BlockSpec(memory_space=pltpu.VMEM),
        scratch_shapes=[
            pltpu.VMEM((N_DEV, d_cat, d_ff), jnp.bfloat16),
            pltpu.SemaphoreType.DMA((N_DEV,)),
            pltpu.SemaphoreType.DMA((N_DEV,)),
        ],
        compiler_params=pltpu.CompilerParams(collective_id=0),
    )(x, router_W, route_idx, ew16, sw16)
